# baseline (speedup 1.0000x reference)
import dataclasses
import functools

import jax
import jax.numpy as jnp
from jax import lax
from jax.experimental import pallas as pl
from jax.experimental.pallas import tpu as pltpu
from jax.experimental.pallas import tpu_sc as plsc

_B = 16384
_D = 64
_NC = 2
_NS = 16
_NW = _NC * _NS
_CHUNK = _B // _NW
_L = 16


def _sc_body(utab_hbm, ptab_hbm, users_hbm, places_hbm, out_hbm,
             uidx_v, pidx_v, urows_v, prows_v, part_v, out_v, sem_u, sem_p):
    wid = lax.axis_index("s") * _NC + lax.axis_index("c")
    base = wid * _CHUNK

    pltpu.sync_copy(users_hbm.at[pl.ds(base, _CHUNK)], uidx_v)
    pltpu.sync_copy(places_hbm.at[pl.ds(base, _CHUNK)], pidx_v)
    cu = pltpu.async_copy(utab_hbm.at[uidx_v], urows_v, sem_u)
    cp = pltpu.async_copy(ptab_hbm.at[pidx_v], prows_v, sem_p)
    cu.wait()
    cp.wait()

    @pl.loop(0, _CHUNK)
    def _(r):
        acc = urows_v[r, pl.ds(0, _L)] * prows_v[r, pl.ds(0, _L)]
        for k in range(1, _D // _L):
            acc = acc + urows_v[r, pl.ds(k * _L, _L)] * prows_v[r, pl.ds(k * _L, _L)]
        part_v[r, pl.ds(0, _L)] = acc

    lanes = jnp.arange(_L, dtype=jnp.int32)

    @pl.loop(0, _CHUNK // _L)
    def _(t):
        rows = t * _L + lanes
        tot = plsc.load_gather(part_v, [rows, jnp.zeros((_L,), jnp.int32)])
        for d in range(1, _L):
            tot = tot + plsc.load_gather(part_v, [rows, jnp.full((_L,), d, jnp.int32)])
        out_v[pl.ds(t * _L, _L)] = tot

    pltpu.sync_copy(out_v, out_hbm.at[pl.ds(base, _CHUNK)])


@jax.jit
def _mfm_sc(users, places, user_table, place_table):
    mesh = plsc.VectorSubcoreMesh(core_axis_name="c", subcore_axis_name="s")
    cp = pltpu.CompilerParams()
    if "needs_layout_passes" in pltpu.CompilerParams.__dataclass_fields__:
        cp = dataclasses.replace(cp, needs_layout_passes=False)
    if "use_tc_tiling_on_sc" in pltpu.CompilerParams.__dataclass_fields__:
        cp = dataclasses.replace(cp, use_tc_tiling_on_sc=False)
    f = pl.kernel(
        _sc_body,
        out_type=jax.ShapeDtypeStruct((_B,), jnp.float32),
        mesh=mesh,
        compiler_params=cp,
        scratch_types=[
            pltpu.VMEM((_CHUNK,), jnp.int32),
            pltpu.VMEM((_CHUNK,), jnp.int32),
            pltpu.VMEM((_CHUNK, _D), jnp.float32),
            pltpu.VMEM((_CHUNK, _D), jnp.float32),
            pltpu.VMEM((_CHUNK, 17), jnp.float32),
            pltpu.VMEM((_CHUNK,), jnp.float32),
            pltpu.SemaphoreType.DMA,
            pltpu.SemaphoreType.DMA,
        ],
    )
    return f(user_table, place_table, users, places)


def kernel(users, places, user_table, place_table):
    return _mfm_sc(users.astype(jnp.int32), places.astype(jnp.int32),
                   user_table, place_table)


# speedup vs baseline: 1.0480x; 1.0480x over previous
import dataclasses
import functools

import jax
import jax.numpy as jnp
from jax import lax
from jax.experimental import pallas as pl
from jax.experimental.pallas import tpu as pltpu
from jax.experimental.pallas import tpu_sc as plsc

_B = 16384
_D = 64
_NROWS = 1000000
_NC = 2
_NS = 16
_NW = _NC * _NS
_CHUNK = _B // _NW
_L = 16
_W = 384
_NFULL = _NROWS // _W
_TAIL0 = _NFULL * _W
_NTAIL = _NROWS - _TAIL0
_NWIN = _NFULL + 1
_KMAX = (_NWIN + _NW - 1) // _NW
_PST = 17


def _gather_one(bufs, tail_v, is_tail, r0, idx16, m, pos16, stage_v, row_v,
                out_hbm, sem_w):
    lanes = jnp.arange(_L, dtype=jnp.int32)
    rloc = idx16 - r0
    rc = jnp.minimum(jnp.maximum(rloc, 0), _W - 1)
    rt = jnp.minimum(jnp.maximum(idx16 - _TAIL0, 0), _NTAIL - 1)

    @pl.when(jnp.logical_not(is_tail))
    def _():
        for a in range(8):
            for b in range(8):
                v = plsc.load_gather(bufs[a], [jnp.full((_L,), b, jnp.int32), rc],
                                     mask=m)
                stage_v[pl.ds((a * 8 + b) * _PST, _L)] = v

    @pl.when(is_tail)
    def _():
        for a in range(8):
            for b in range(8):
                d = a * 8 + b
                v = plsc.load_gather(tail_v, [rt * _D + d], mask=m)
                stage_v[pl.ds(d * _PST, _L)] = v

    mi = jnp.where(m, 1, 0).astype(jnp.int32)
    for j in range(_L):
        @pl.when(mi[j] == 1)
        def _():
            for k in range(_D // _L):
                row_v[pl.ds(j * 128 + k * _L, _L)] = plsc.load_gather(
                    stage_v, [(16 * k + lanes) * _PST + j])
            pltpu.async_copy(row_v.at[pl.ds(j * 128, 128)],
                             out_hbm.at[pos16[j]], sem_w)

    pc = plsc.all_reduce_population_count(m)[0]

    def drain(_, c):
        pltpu.make_async_copy(out_hbm.at[0], row_v.at[pl.ds(0, 128)],
                              sem_w).wait()
        return c

    lax.fori_loop(0, pc, drain, 0)


def _k1_body(utab_hbm, ptab_hbm, users_hbm, places_hbm, tailu_hbm, tailp_hbm,
             gu_hbm, gp_hbm,
             uidx_v, pidx_v, ulist_v, plist_v, ubuf0, ubuf1, ubuf2, ubuf3,
             ubuf4, ubuf5, ubuf6, ubuf7, pbuf0, pbuf1, pbuf2, pbuf3, pbuf4,
             pbuf5, pbuf6, pbuf7, tailu_v, tailp_v, stage_v, row_v,
             sem_s, sem_w):
    wid = lax.axis_index("s") * _NC + lax.axis_index("c")
    ubufs = [ubuf0, ubuf1, ubuf2, ubuf3, ubuf4, ubuf5, ubuf6, ubuf7]
    pbufs = [pbuf0, pbuf1, pbuf2, pbuf3, pbuf4, pbuf5, pbuf6, pbuf7]
    lanes = jnp.arange(_L, dtype=jnp.int32)

    pltpu.sync_copy(users_hbm, uidx_v)
    pltpu.sync_copy(places_hbm, pidx_v)
    pltpu.sync_copy(tailu_hbm, tailu_v)
    pltpu.sync_copy(tailp_hbm, tailp_v)

    def compact(g, carry):
        ou, op = carry
        pos16 = g * _L + lanes
        iu = uidx_v[pl.ds(g * _L, _L)]
        ip = pidx_v[pl.ds(g * _L, _L)]
        mu = (jnp.minimum(iu // _W, _NFULL) & (_NW - 1)) == wid
        mp = (jnp.minimum(ip // _W, _NFULL) & (_NW - 1)) == wid
        plsc.store_compressed(ulist_v.at[pl.ds(ou, _L)], pos16, mask=mu)
        plsc.store_compressed(plist_v.at[pl.ds(op, _L)], pos16, mask=mp)
        return (ou + plsc.all_reduce_population_count(mu)[0],
                op + plsc.all_reduce_population_count(mp)[0])

    ucount, pcount = lax.fori_loop(0, _B // _L, compact, (0, 0))
    ug = (ucount + _L - 1) // _L
    pg = (pcount + _L - 1) // _L

    @pl.loop(0, _KMAX)
    def _(k):
        win = k * _NW + wid

        @pl.when(win < _NWIN)
        def _():
            is_tail = win == _NFULL
            r0 = win * _W

            @pl.when(jnp.logical_not(is_tail))
            def _():
                copies = []
                for a in range(8):
                    copies.append(pltpu.async_copy(
                        utab_hbm.at[pl.ds(a * 8, 8), pl.ds(r0, _W)],
                        ubufs[a], sem_s))
                    copies.append(pltpu.async_copy(
                        ptab_hbm.at[pl.ds(a * 8, 8), pl.ds(r0, _W)],
                        pbufs[a], sem_s))
                for c in copies:
                    c.wait()

            def serve(lst, idxs, cnt, ngrp, bufs, tail_v, ghbm):
                def grp(g, npend):
                    raw = plsc.load_gather(
                        lst, [jnp.minimum(g * _L + lanes, _B - 1)])
                    pos16 = jnp.minimum(jnp.maximum(raw, 0), _B - 1)
                    idx16 = plsc.load_gather(idxs, [pos16])
                    m = ((idx16 >= r0)
                         & (idx16 < jnp.where(is_tail, _NROWS, r0 + _W))
                         & ((g * _L + lanes) < cnt))

                    @pl.when(jnp.any(m))
                    def _():
                        _gather_one(bufs, tail_v, is_tail, r0, idx16, m,
                                    pos16, stage_v, row_v, ghbm, sem_w)
                    return npend

                lax.fori_loop(0, ngrp, grp, 0)

            serve(ulist_v, uidx_v, ucount, ug, ubufs, tailu_v, gu_hbm)
            serve(plist_v, pidx_v, pcount, pg, pbufs, tailp_v, gp_hbm)


def _k2_body(gu_hbm, gp_hbm, out_hbm, gu_v, gp_v, part_v, out_v, sem):
    wid = lax.axis_index("s") * _NC + lax.axis_index("c")
    base = wid * _CHUNK
    lanes = jnp.arange(_L, dtype=jnp.int32) * _PST

    @pl.loop(0, _CHUNK // 128)
    def _(r):
        b0 = base + r * 128
        cu = pltpu.async_copy(gu_hbm.at[pl.ds(b0, 128), :], gu_v, sem)
        cp = pltpu.async_copy(gp_hbm.at[pl.ds(b0, 128), :], gp_v, sem)
        cu.wait()
        cp.wait()

        @pl.loop(0, 128)
        def _(i):
            acc = gu_v[i, pl.ds(0, _L)] * gp_v[i, pl.ds(0, _L)]
            for k in range(1, _D // _L):
                acc = acc + gu_v[i, pl.ds(k * _L, _L)] * gp_v[i, pl.ds(k * _L, _L)]
            part_v[pl.ds(i * _PST, _L)] = acc

        @pl.loop(0, 128 // _L)
        def _(t):
            rows = t * (_L * _PST) + lanes
            tot = plsc.load_gather(part_v, [rows])
            for d in range(1, _L):
                tot = tot + plsc.load_gather(part_v, [rows + d])
            out_v[pl.ds(r * 128 + t * _L, _L)] = tot

    pltpu.sync_copy(out_v, out_hbm.at[pl.ds(base, _CHUNK)])


def _compiler_params():
    cp = pltpu.CompilerParams()
    if "needs_layout_passes" in pltpu.CompilerParams.__dataclass_fields__:
        cp = dataclasses.replace(cp, needs_layout_passes=False)
    if "use_tc_tiling_on_sc" in pltpu.CompilerParams.__dataclass_fields__:
        cp = dataclasses.replace(cp, use_tc_tiling_on_sc=True)
    return cp


@jax.jit
def _mfm_sc(users, places, user_table, place_table):
    mesh = plsc.VectorSubcoreMesh(core_axis_name="c", subcore_axis_name="s")
    cp = _compiler_params()

    k1 = pl.kernel(
        _k1_body,
        out_type=(jax.ShapeDtypeStruct((_B, 128), jnp.float32),
                  jax.ShapeDtypeStruct((_B, 128), jnp.float32)),
        mesh=mesh,
        compiler_params=cp,
        scratch_types=(
            [pltpu.VMEM((_B,), jnp.int32), pltpu.VMEM((_B,), jnp.int32),
             pltpu.VMEM((_B,), jnp.int32), pltpu.VMEM((_B,), jnp.int32)]
            + [pltpu.VMEM((8, _W), jnp.float32) for _ in range(16)]
            + [pltpu.VMEM((_NTAIL * _D,), jnp.float32),
               pltpu.VMEM((_NTAIL * _D,), jnp.float32),
               pltpu.VMEM((_D * _PST,), jnp.float32),
               pltpu.VMEM((_L * 128,), jnp.float32),
               pltpu.SemaphoreType.DMA,
               pltpu.SemaphoreType.DMA]
        ),
    )

    k2 = pl.kernel(
        _k2_body,
        out_type=jax.ShapeDtypeStruct((_B,), jnp.float32),
        mesh=mesh,
        compiler_params=cp,
        scratch_types=[
            pltpu.VMEM((128, 128), jnp.float32),
            pltpu.VMEM((128, 128), jnp.float32),
            pltpu.VMEM((128 * _PST,), jnp.float32),
            pltpu.VMEM((_CHUNK,), jnp.float32),
            pltpu.SemaphoreType.DMA,
        ],
    )

    utab2 = user_table.T
    ptab2 = place_table.T
    tailu = user_table[_TAIL0:, :].reshape(-1)
    tailp = place_table[_TAIL0:, :].reshape(-1)
    gu, gp = k1(utab2, ptab2, users, places, tailu, tailp)
    return k2(gu, gp)


def kernel(users, places, user_table, place_table):
    return _mfm_sc(users.astype(jnp.int32), places.astype(jnp.int32),
                   user_table, place_table)


# speedup vs baseline: 1.8267x; 1.7430x over previous
import dataclasses
import functools

import jax
import jax.numpy as jnp
from jax import lax
from jax.experimental import pallas as pl
from jax.experimental.pallas import tpu as pltpu
from jax.experimental.pallas import tpu_sc as plsc

_B = 16384
_D = 64
_NROWS = 1000000
_NC = 2
_NS = 16
_NW = _NC * _NS
_CHUNK = _B // _NW
_L = 16
_W = 256
_NFULL = _NROWS // _W
_TAIL0 = _NFULL * _W
_NTAIL = _NROWS - _TAIL0
_NWIN = _NFULL + 1
_KMAX = (_NWIN + _NW - 1) // _NW
_PST = 17


def _gather_one(bufs, tail_v, is_tail, r0, idx16, m, pos16, stage_v, row_v,
                out_hbm, sem_w):
    lanes = jnp.arange(_L, dtype=jnp.int32)
    rloc = idx16 - r0
    rc = jnp.minimum(jnp.maximum(rloc, 0), _W - 1)
    rt = jnp.minimum(jnp.maximum(idx16 - _TAIL0, 0), _NTAIL - 1)

    @pl.when(jnp.logical_not(is_tail))
    def _():
        for a in range(8):
            for b in range(8):
                v = plsc.load_gather(bufs[a], [jnp.full((_L,), b, jnp.int32), rc],
                                     mask=m)
                stage_v[pl.ds((a * 8 + b) * _PST, _L)] = v

    @pl.when(is_tail)
    def _():
        for a in range(8):
            for b in range(8):
                d = a * 8 + b
                v = plsc.load_gather(tail_v, [rt * _D + d], mask=m)
                stage_v[pl.ds(d * _PST, _L)] = v

    mi = jnp.where(m, 1, 0).astype(jnp.int32)
    for j in range(_L):
        @pl.when(mi[j] == 1)
        def _():
            for k in range(_D // _L):
                row_v[pl.ds(j * 128 + k * _L, _L)] = plsc.load_gather(
                    stage_v, [(16 * k + lanes) * _PST + j])
            pltpu.async_copy(row_v.at[pl.ds(j * 128, 128)],
                             out_hbm.at[pos16[j]], sem_w)

    return plsc.all_reduce_population_count(m)[0]


def _k1_body(utab_hbm, ptab_hbm, users_hbm, places_hbm, tailu_hbm, tailp_hbm,
             gu_hbm, gp_hbm,
             uidx_v, pidx_v, ulist_v, plist_v, ubuf0, ubuf1, ubuf2, ubuf3,
             ubuf4, ubuf5, ubuf6, ubuf7, pbuf0, pbuf1, pbuf2, pbuf3, pbuf4,
             pbuf5, pbuf6, pbuf7, tailu_v, tailp_v, wpos_v, stage_v, row_v,
             sem_s, sem_w):
    wid = lax.axis_index("s") * _NC + lax.axis_index("c")
    ubufs = [ubuf0, ubuf1, ubuf2, ubuf3, ubuf4, ubuf5, ubuf6, ubuf7]
    pbufs = [pbuf0, pbuf1, pbuf2, pbuf3, pbuf4, pbuf5, pbuf6, pbuf7]
    lanes = jnp.arange(_L, dtype=jnp.int32)

    pltpu.sync_copy(users_hbm, uidx_v)
    pltpu.sync_copy(places_hbm, pidx_v)
    pltpu.sync_copy(tailu_hbm, tailu_v)
    pltpu.sync_copy(tailp_hbm, tailp_v)

    def compact(g, carry):
        ou, op = carry
        pos16 = g * _L + lanes
        iu = uidx_v[pl.ds(g * _L, _L)]
        ip = pidx_v[pl.ds(g * _L, _L)]
        mu = (jnp.minimum(iu // _W, _NFULL) & (_NW - 1)) == wid
        mp = (jnp.minimum(ip // _W, _NFULL) & (_NW - 1)) == wid
        plsc.store_compressed(ulist_v.at[pl.ds(ou, _L)], pos16, mask=mu)
        plsc.store_compressed(plist_v.at[pl.ds(op, _L)], pos16, mask=mp)
        return (ou + plsc.all_reduce_population_count(mu)[0],
                op + plsc.all_reduce_population_count(mp)[0])

    ucount, pcount = lax.fori_loop(0, _B // _L, compact, (0, 0))
    ug = (ucount + _L - 1) // _L
    pg = (pcount + _L - 1) // _L

    def drain(n):
        def one(_, c):
            pltpu.make_async_copy(gu_hbm.at[0], row_v.at[pl.ds(0, 128)],
                                  sem_w).wait()
            return c
        lax.fori_loop(0, n, one, 0)

    def window(k, pend):
        win = jnp.minimum(k * _NW + wid, _NWIN - 1)
        is_tail = win == _NFULL
        r0 = win * _W

        @pl.when(jnp.logical_not(is_tail))
        def _():
            copies = []
            for a in range(8):
                copies.append(pltpu.async_copy(
                    utab_hbm.at[pl.ds(a * 8, 8), pl.ds(r0, _W)],
                    ubufs[a], sem_s))
                copies.append(pltpu.async_copy(
                    ptab_hbm.at[pl.ds(a * 8, 8), pl.ds(r0, _W)],
                    pbufs[a], sem_s))
            for c in copies:
                c.wait()

        def serve(lst, idxs, cnt, ngrp, bufs, tail_v, ghbm, pend):
            def comp(g, off):
                raw = plsc.load_gather(
                    lst, [jnp.minimum(g * _L + lanes, _B - 1)])
                pos16 = jnp.minimum(jnp.maximum(raw, 0), _B - 1)
                idx16 = plsc.load_gather(idxs, [pos16])
                m = ((idx16 >= r0)
                     & (idx16 < jnp.where(is_tail, _NROWS, r0 + _W))
                     & ((g * _L + lanes) < cnt))
                plsc.store_compressed(wpos_v.at[pl.ds(off, _L)], pos16,
                                      mask=m)
                return off + plsc.all_reduce_population_count(m)[0]

            wcnt = lax.fori_loop(0, ngrp, comp, 0)

            def grp(g, pd):
                drain(pd)
                raw = plsc.load_gather(
                    wpos_v, [jnp.minimum(g * _L + lanes, _B - 1)])
                pos16 = jnp.minimum(jnp.maximum(raw, 0), _B - 1)
                idx16 = plsc.load_gather(idxs, [pos16])
                m = (g * _L + lanes) < wcnt
                return _gather_one(bufs, tail_v, is_tail, r0, idx16, m,
                                   pos16, stage_v, row_v, ghbm, sem_w)

            return lax.fori_loop(0, (wcnt + _L - 1) // _L, grp, pend)

        pend = serve(ulist_v, uidx_v, ucount, ug, ubufs, tailu_v, gu_hbm,
                     pend)
        pend = serve(plist_v, pidx_v, pcount, pg, pbufs, tailp_v, gp_hbm,
                     pend)
        return pend

    pend = lax.fori_loop(0, _KMAX, window, 0)
    drain(pend)


def _k2_body(gu_hbm, gp_hbm, out_hbm, gu_v, gp_v, part_v, out_v, sem):
    wid = lax.axis_index("s") * _NC + lax.axis_index("c")
    base = wid * _CHUNK
    lanes = jnp.arange(_L, dtype=jnp.int32) * _PST

    @pl.loop(0, _CHUNK // 128)
    def _(r):
        b0 = base + r * 128
        cu = pltpu.async_copy(gu_hbm.at[pl.ds(b0, 128), :], gu_v, sem)
        cp = pltpu.async_copy(gp_hbm.at[pl.ds(b0, 128), :], gp_v, sem)
        cu.wait()
        cp.wait()

        @pl.loop(0, 128)
        def _(i):
            acc = gu_v[i, pl.ds(0, _L)] * gp_v[i, pl.ds(0, _L)]
            for k in range(1, _D // _L):
                acc = acc + gu_v[i, pl.ds(k * _L, _L)] * gp_v[i, pl.ds(k * _L, _L)]
            part_v[pl.ds(i * _PST, _L)] = acc

        @pl.loop(0, 128 // _L)
        def _(t):
            rows = t * (_L * _PST) + lanes
            tot = plsc.load_gather(part_v, [rows])
            for d in range(1, _L):
                tot = tot + plsc.load_gather(part_v, [rows + d])
            out_v[pl.ds(r * 128 + t * _L, _L)] = tot

    pltpu.sync_copy(out_v, out_hbm.at[pl.ds(base, _CHUNK)])


def _compiler_params():
    cp = pltpu.CompilerParams()
    if "needs_layout_passes" in pltpu.CompilerParams.__dataclass_fields__:
        cp = dataclasses.replace(cp, needs_layout_passes=False)
    if "use_tc_tiling_on_sc" in pltpu.CompilerParams.__dataclass_fields__:
        cp = dataclasses.replace(cp, use_tc_tiling_on_sc=True)
    return cp


@jax.jit
def _mfm_sc(users, places, user_table, place_table):
    mesh = plsc.VectorSubcoreMesh(core_axis_name="c", subcore_axis_name="s")
    cp = _compiler_params()

    k1 = pl.kernel(
        _k1_body,
        out_type=(jax.ShapeDtypeStruct((_B, 128), jnp.float32),
                  jax.ShapeDtypeStruct((_B, 128), jnp.float32)),
        mesh=mesh,
        compiler_params=cp,
        scratch_types=(
            [pltpu.VMEM((_B,), jnp.int32), pltpu.VMEM((_B,), jnp.int32),
             pltpu.VMEM((_B,), jnp.int32), pltpu.VMEM((_B,), jnp.int32)]
            + [pltpu.VMEM((8, _W), jnp.float32) for _ in range(16)]
            + [pltpu.VMEM((_NTAIL * _D,), jnp.float32),
               pltpu.VMEM((_NTAIL * _D,), jnp.float32),
               pltpu.VMEM((_B,), jnp.int32),
               pltpu.VMEM((_D * _PST,), jnp.float32),
               pltpu.VMEM((_L * 128,), jnp.float32),
               pltpu.SemaphoreType.DMA,
               pltpu.SemaphoreType.DMA]
        ),
    )

    k2 = pl.kernel(
        _k2_body,
        out_type=jax.ShapeDtypeStruct((_B,), jnp.float32),
        mesh=mesh,
        compiler_params=cp,
        scratch_types=[
            pltpu.VMEM((128, 128), jnp.float32),
            pltpu.VMEM((128, 128), jnp.float32),
            pltpu.VMEM((128 * _PST,), jnp.float32),
            pltpu.VMEM((_CHUNK,), jnp.float32),
            pltpu.SemaphoreType.DMA,
        ],
    )

    utab2 = user_table.T
    ptab2 = place_table.T
    tailu = user_table[_TAIL0:, :].reshape(-1)
    tailp = place_table[_TAIL0:, :].reshape(-1)
    gu, gp = k1(utab2, ptab2, users, places, tailu, tailp)
    return k2(gu, gp)


def kernel(users, places, user_table, place_table):
    return _mfm_sc(users.astype(jnp.int32), places.astype(jnp.int32),
                   user_table, place_table)
